# Initial kernel scaffold; baseline (speedup 1.0000x reference)
import jax
import jax.numpy as jnp
from jax.experimental import pallas as pl


def kernel(x, edge_index, batch, W1, b1, W2, b2, Wfc, bfc):
    raise NotImplementedError("write your pallas kernel here")


# speedup vs baseline: 11.1843x; 11.1843x over previous
import functools

import jax
import jax.numpy as jnp
from jax import lax
from jax.experimental import pallas as pl
from jax.experimental.pallas import tpu as pltpu
from jax.experimental.pallas import tpu_sc as plsc

N = 10000
NP = 10240
E = 320000
D = 128
H = 128
C = 26
G = 64
NC, NS = 2, 16
NW = NC * NS
EPW = E // NW
CH = 80
NCH = EPW // CH
RPS = NP // NS
DW = 8
RB = 256
NRB = NP // RB

_sc_mesh = plsc.VectorSubcoreMesh(core_axis_name="c", subcore_axis_name="s")


def _sc_pass_body(gather, src_hbm_or_none, y_hbm, zeros_hbm, ones_hbm,
                  dst_hbm, out0_hbm, out1_hbm, srcv, dstv, rows, acc, sem, c, s):
    for ks in range(NS):
        @pl.when(s == ks)
        def _():
            for k in range(RPS // CH):
                off = ks * RPS + k * CH

                @pl.when(c == 0)
                def _():
                    pltpu.sync_copy(y_hbm.at[pl.ds(off, CH)], rows)
                    pltpu.sync_copy(rows, acc.at[pl.ds(off, CH)])

                @pl.when(c != 0)
                def _():
                    pltpu.sync_copy(zeros_hbm.at[pl.ds(off, CH)], rows)
                    pltpu.sync_copy(rows, acc.at[pl.ds(off, CH)])

    if not gather:
        pltpu.sync_copy(ones_hbm, rows)
    plsc.subcore_barrier()
    base = (c * NS + s) * EPW

    def body(i, carry):
        pltpu.sync_copy(dst_hbm.at[pl.ds(base + i * CH, CH)], dstv)
        if gather:
            pltpu.sync_copy(src_hbm_or_none.at[pl.ds(base + i * CH, CH)], srcv)
            pltpu.async_copy(y_hbm.at[srcv], rows, sem).wait()
        pltpu.sync_copy(rows, acc.at[dstv], add=True)
        return carry

    lax.fori_loop(0, NCH, body, 0)
    plsc.subcore_barrier()
    for ks in range(NS):
        @pl.when(s == ks)
        def _():
            for k in range(RPS // CH):
                off = ks * RPS + k * CH
                pltpu.sync_copy(acc.at[pl.ds(off, CH)], rows)

                @pl.when(c == 0)
                def _():
                    pltpu.sync_copy(rows, out0_hbm.at[pl.ds(off, CH)])

                @pl.when(c != 0)
                def _():
                    pltpu.sync_copy(rows, out1_hbm.at[pl.ds(off, CH)])


def _deg_body(dst_hbm, zeros_hbm, ones_hbm, out0_hbm, out1_hbm,
              dstv, rows, acc):
    c = lax.axis_index("c")
    s = lax.axis_index("s")
    _sc_pass_body(False, None, zeros_hbm, zeros_hbm, ones_hbm, dst_hbm,
                  out0_hbm, out1_hbm, None, dstv, rows, acc, None, c, s)


def _sc_deg(dst_flat, zeros2, ones2):
    p0, p1 = pl.kernel(
        _deg_body,
        out_type=[jax.ShapeDtypeStruct((NP, 128), jnp.float32),
                  jax.ShapeDtypeStruct((NP, 128), jnp.float32)],
        mesh=_sc_mesh,
        scratch_types=[
            pltpu.VMEM((CH,), jnp.int32),
            pltpu.VMEM((CH, 128), jnp.float32),
            pltpu.VMEM_SHARED((NP, 128), jnp.float32),
        ],
    )(dst_flat, zeros2, ones2)
    return jnp.stack([p0, p1])


def _agg_body(y_hbm, zeros_hbm, src_hbm, dst_hbm, out0_hbm, out1_hbm,
              srcv, dstv, rows, acc, sem):
    c = lax.axis_index("c")
    s = lax.axis_index("s")
    _sc_pass_body(True, src_hbm, y_hbm, zeros_hbm, None, dst_hbm,
                  out0_hbm, out1_hbm, srcv, dstv, rows, acc, sem, c, s)


def _sc_agg(y, zeros2, src_flat, dst_flat):
    p0, p1 = pl.kernel(
        _agg_body,
        out_type=[jax.ShapeDtypeStruct((NP, H), jnp.float32),
                  jax.ShapeDtypeStruct((NP, H), jnp.float32)],
        mesh=_sc_mesh,
        scratch_types=[
            pltpu.VMEM((CH,), jnp.int32),
            pltpu.VMEM((CH,), jnp.int32),
            pltpu.VMEM((CH, H), jnp.float32),
            pltpu.VMEM_SHARED((NP, H), jnp.float32),
            pltpu.SemaphoreType.DMA,
        ],
    )(y, zeros2, src_flat, dst_flat)
    return jnp.stack([p0, p1])


def _dinv_block(degp_ref):
    deg = degp_ref[0][:, 0:1] + degp_ref[1][:, 0:1]
    return lax.rsqrt(deg + 1.0)


def _tc1_body(degp_ref, x_ref, w_ref, y_ref):
    dinv = _dinv_block(degp_ref)
    y_ref[...] = dinv * jnp.dot(x_ref[...], w_ref[...],
                                preferred_element_type=jnp.float32)


def _tc1(deg_parts, x_pad, W1):
    return pl.pallas_call(
        _tc1_body,
        grid=(NRB,),
        in_specs=[
            pl.BlockSpec((NC, RB, 128), lambda i: (0, i, 0)),
            pl.BlockSpec((RB, D), lambda i: (i, 0)),
            pl.BlockSpec((D, H), lambda i: (0, 0)),
        ],
        out_specs=pl.BlockSpec((RB, H), lambda i: (i, 0)),
        out_shape=jax.ShapeDtypeStruct((NP, H), jnp.float32),
    )(deg_parts, x_pad, W1)


def _tc2_body(aggp_ref, degp_ref, b_ref, w_ref, y2_ref):
    dinv = _dinv_block(degp_ref)
    h = jnp.maximum(dinv * (aggp_ref[0] + aggp_ref[1]) + b_ref[...], 0.0)
    y2_ref[...] = dinv * jnp.dot(h, w_ref[...],
                                 preferred_element_type=jnp.float32)


def _tc2(agg_parts, deg_parts, b_row, W2):
    return pl.pallas_call(
        _tc2_body,
        grid=(NRB,),
        in_specs=[
            pl.BlockSpec((NC, RB, H), lambda i: (0, i, 0)),
            pl.BlockSpec((NC, RB, 128), lambda i: (0, i, 0)),
            pl.BlockSpec((1, H), lambda i: (0, 0)),
            pl.BlockSpec((H, H), lambda i: (0, 0)),
        ],
        out_specs=pl.BlockSpec((RB, H), lambda i: (i, 0)),
        out_shape=jax.ShapeDtypeStruct((NP, H), jnp.float32),
    )(agg_parts, deg_parts, b_row, W2)


def _tc3_body(aggp_ref, degp_ref, b_ref, batch_ref, wfc_ref, bfc_ref,
              out_ref, sums, cnts):
    i = pl.program_id(0)

    @pl.when(i == 0)
    def _():
        sums[...] = jnp.zeros_like(sums)
        cnts[...] = jnp.zeros_like(cnts)

    dinv = _dinv_block(degp_ref)
    h = jnp.maximum(dinv * (aggp_ref[0] + aggp_ref[1]) + b_ref[...], 0.0)
    gid = lax.broadcasted_iota(jnp.int32, (RB, 128), 1)
    onehot = (batch_ref[...] == gid).astype(jnp.float32)
    sums[...] += lax.dot_general(onehot, h, (((0,), (0,)), ((), ())),
                                 preferred_element_type=jnp.float32)
    cnts[...] += lax.dot_general(onehot, jnp.ones((RB, 1), jnp.float32),
                                 (((0,), (0,)), ((), ())),
                                 preferred_element_type=jnp.float32)

    @pl.when(i == NRB - 1)
    def _():
        pooled = sums[...] / jnp.maximum(cnts[...], 1.0)
        out_ref[...] = jnp.dot(pooled, wfc_ref[...],
                               preferred_element_type=jnp.float32) + bfc_ref[...]


def _tc3(agg_parts, deg_parts, b_row, batch2, wfc_pad, bfc_row):
    return pl.pallas_call(
        _tc3_body,
        grid=(NRB,),
        in_specs=[
            pl.BlockSpec((NC, RB, H), lambda i: (0, i, 0)),
            pl.BlockSpec((NC, RB, 128), lambda i: (0, i, 0)),
            pl.BlockSpec((1, H), lambda i: (0, 0)),
            pl.BlockSpec((RB, 1), lambda i: (i, 0)),
            pl.BlockSpec((H, 128), lambda i: (0, 0)),
            pl.BlockSpec((1, 128), lambda i: (0, 0)),
        ],
        out_specs=pl.BlockSpec((128, 128), lambda i: (0, 0)),
        out_shape=jax.ShapeDtypeStruct((128, 128), jnp.float32),
        scratch_shapes=[
            pltpu.VMEM((128, 128), jnp.float32),
            pltpu.VMEM((128, 1), jnp.float32),
        ],
    )(agg_parts, deg_parts, b_row, batch2, wfc_pad, bfc_row)


def kernel(x, edge_index, batch, W1, b1, W2, b2, Wfc, bfc):
    x = x.astype(jnp.float32)
    src_flat = edge_index[0].astype(jnp.int32)
    dst_flat = edge_index[1].astype(jnp.int32)
    batch2 = jnp.pad(batch.astype(jnp.int32), (0, NP - N),
                     constant_values=G).reshape(NP, 1)
    x_pad = jnp.pad(x, ((0, NP - N), (0, 0)))
    zeros2 = jnp.zeros((NP, H), jnp.float32)
    ones2 = jnp.ones((CH, 128), jnp.float32)
    wfc_pad = jnp.pad(Wfc.astype(jnp.float32), ((0, 0), (0, 128 - C)))
    bfc_row = jnp.pad(bfc.astype(jnp.float32), (0, 128 - C)).reshape(1, 128)
    b1_row = b1.astype(jnp.float32).reshape(1, H)
    b2_row = b2.astype(jnp.float32).reshape(1, H)

    deg_parts = _sc_deg(dst_flat, zeros2, ones2)
    y1 = _tc1(deg_parts, x_pad, W1.astype(jnp.float32))
    agg1 = _sc_agg(y1, zeros2, src_flat, dst_flat)
    y2 = _tc2(agg1, deg_parts, b1_row, W2.astype(jnp.float32))
    agg2 = _sc_agg(y2, zeros2, src_flat, dst_flat)
    outp = _tc3(agg2, deg_parts, b2_row, batch2, wfc_pad, bfc_row)
    return outp[:G, :C]


# speedup vs baseline: 15.3981x; 1.3768x over previous
import functools

import jax
import jax.numpy as jnp
from jax import lax
from jax.experimental import pallas as pl
from jax.experimental.pallas import tpu as pltpu
from jax.experimental.pallas import tpu_sc as plsc

N = 10000
NP = 10240
E = 320000
D = 128
H = 128
C = 26
G = 64
NC, NS = 2, 16
NW = NC * NS
EPW = E // NW
CH = 80
NCH = EPW // CH
RPS = NP // NS
DW = 8
RB = 256
NRB = NP // RB

_sc_mesh = plsc.VectorSubcoreMesh(core_axis_name="c", subcore_axis_name="s")


def _sc_pass_body(gather, src_hbm, y_hbm, zeros_hbm, ones_hbm, dst_hbm,
                  out0_hbm, out1_hbm, bufs, c, s):
    if gather:
        srcvA, srcvB, dstvA, dstvB, rowsA, rowsB, acc, semA, semB = bufs
    else:
        dstvA, dstvB, rowsA, acc = bufs
    for ks in range(NS):
        @pl.when(s == ks)
        def _():
            for k in range(RPS // CH):
                off = ks * RPS + k * CH

                @pl.when(c == 0)
                def _():
                    pltpu.sync_copy(y_hbm.at[pl.ds(off, CH)], rowsA)
                    pltpu.sync_copy(rowsA, acc.at[pl.ds(off, CH)])

                @pl.when(c != 0)
                def _():
                    pltpu.sync_copy(zeros_hbm.at[pl.ds(off, CH)], rowsA)
                    pltpu.sync_copy(rowsA, acc.at[pl.ds(off, CH)])

    if not gather:
        pltpu.sync_copy(ones_hbm, rowsA)
    plsc.subcore_barrier()
    base = (c * NS + s) * EPW

    if gather:
        def load(ch, srcv, dstv):
            pltpu.sync_copy(src_hbm.at[pl.ds(base + ch * CH, CH)], srcv)
            pltpu.sync_copy(dst_hbm.at[pl.ds(base + ch * CH, CH)], dstv)

        load(0, srcvA, dstvA)
        pltpu.async_copy(y_hbm.at[srcvA], rowsA, semA)

        def body2(j, carry):
            load(2 * j + 1, srcvB, dstvB)
            pltpu.async_copy(y_hbm.at[srcvB], rowsB, semB)
            pltpu.make_async_copy(y_hbm.at[srcvA], rowsA, semA).wait()
            pltpu.sync_copy(rowsA, acc.at[dstvA], add=True)
            load(2 * j + 2, srcvA, dstvA)
            pltpu.async_copy(y_hbm.at[srcvA], rowsA, semA)
            pltpu.make_async_copy(y_hbm.at[srcvB], rowsB, semB).wait()
            pltpu.sync_copy(rowsB, acc.at[dstvB], add=True)
            return carry

        lax.fori_loop(0, (NCH - 1) // 2, body2, 0)
        pltpu.make_async_copy(y_hbm.at[srcvA], rowsA, semA).wait()
        pltpu.sync_copy(rowsA, acc.at[dstvA], add=True)
    else:
        def loadd(ch, dstv):
            pltpu.sync_copy(dst_hbm.at[pl.ds(base + ch * CH, CH)], dstv)

        loadd(0, dstvA)

        def body2(j, carry):
            loadd(2 * j + 1, dstvB)
            pltpu.sync_copy(rowsA, acc.at[dstvA], add=True)
            loadd(2 * j + 2, dstvA)
            pltpu.sync_copy(rowsA, acc.at[dstvB], add=True)
            return carry

        lax.fori_loop(0, (NCH - 1) // 2, body2, 0)
        pltpu.sync_copy(rowsA, acc.at[dstvA], add=True)

    plsc.subcore_barrier()
    for ks in range(NS):
        @pl.when(s == ks)
        def _():
            for k in range(RPS // CH):
                off = ks * RPS + k * CH
                pltpu.sync_copy(acc.at[pl.ds(off, CH)], rowsA)

                @pl.when(c == 0)
                def _():
                    pltpu.sync_copy(rowsA, out0_hbm.at[pl.ds(off, CH)])

                @pl.when(c != 0)
                def _():
                    pltpu.sync_copy(rowsA, out1_hbm.at[pl.ds(off, CH)])


def _deg_body(dst_hbm, zeros_hbm, ones_hbm, out0_hbm, out1_hbm,
              dstvA, dstvB, rowsA, acc):
    c = lax.axis_index("c")
    s = lax.axis_index("s")
    _sc_pass_body(False, None, zeros_hbm, zeros_hbm, ones_hbm, dst_hbm,
                  out0_hbm, out1_hbm, (dstvA, dstvB, rowsA, acc), c, s)


def _sc_deg(dst_flat, zeros2, ones2):
    p0, p1 = pl.kernel(
        _deg_body,
        out_type=[jax.ShapeDtypeStruct((NP, 128), jnp.float32),
                  jax.ShapeDtypeStruct((NP, 128), jnp.float32)],
        mesh=_sc_mesh,
        scratch_types=[
            pltpu.VMEM((CH,), jnp.int32),
            pltpu.VMEM((CH,), jnp.int32),
            pltpu.VMEM((CH, 128), jnp.float32),
            pltpu.VMEM_SHARED((NP, 128), jnp.float32),
        ],
    )(dst_flat, zeros2, ones2)
    return jnp.stack([p0, p1])


def _agg_body(y_hbm, zeros_hbm, src_hbm, dst_hbm, out0_hbm, out1_hbm,
              srcvA, srcvB, dstvA, dstvB, rowsA, rowsB, acc, semA, semB):
    c = lax.axis_index("c")
    s = lax.axis_index("s")
    _sc_pass_body(True, src_hbm, y_hbm, zeros_hbm, None, dst_hbm,
                  out0_hbm, out1_hbm,
                  (srcvA, srcvB, dstvA, dstvB, rowsA, rowsB, acc, semA, semB),
                  c, s)


def _sc_agg(y, zeros2, src_flat, dst_flat):
    p0, p1 = pl.kernel(
        _agg_body,
        out_type=[jax.ShapeDtypeStruct((NP, H), jnp.float32),
                  jax.ShapeDtypeStruct((NP, H), jnp.float32)],
        mesh=_sc_mesh,
        scratch_types=[
            pltpu.VMEM((CH,), jnp.int32),
            pltpu.VMEM((CH,), jnp.int32),
            pltpu.VMEM((CH,), jnp.int32),
            pltpu.VMEM((CH,), jnp.int32),
            pltpu.VMEM((CH, H), jnp.float32),
            pltpu.VMEM((CH, H), jnp.float32),
            pltpu.VMEM_SHARED((NP, H), jnp.float32),
            pltpu.SemaphoreType.DMA,
            pltpu.SemaphoreType.DMA,
        ],
    )(y, zeros2, src_flat, dst_flat)
    return jnp.stack([p0, p1])


def _dinv_block(degp_ref):
    deg = degp_ref[0][:, 0:1] + degp_ref[1][:, 0:1]
    return lax.rsqrt(deg + 1.0)


def _tc1_body(degp_ref, x_ref, w_ref, y_ref):
    dinv = _dinv_block(degp_ref)
    y_ref[...] = dinv * jnp.dot(x_ref[...], w_ref[...],
                                preferred_element_type=jnp.float32)


def _tc1(deg_parts, x_pad, W1):
    return pl.pallas_call(
        _tc1_body,
        grid=(NRB,),
        in_specs=[
            pl.BlockSpec((NC, RB, 128), lambda i: (0, i, 0)),
            pl.BlockSpec((RB, D), lambda i: (i, 0)),
            pl.BlockSpec((D, H), lambda i: (0, 0)),
        ],
        out_specs=pl.BlockSpec((RB, H), lambda i: (i, 0)),
        out_shape=jax.ShapeDtypeStruct((NP, H), jnp.float32),
    )(deg_parts, x_pad, W1)


def _tc2_body(aggp_ref, degp_ref, b_ref, w_ref, y2_ref):
    dinv = _dinv_block(degp_ref)
    h = jnp.maximum(dinv * (aggp_ref[0] + aggp_ref[1]) + b_ref[...], 0.0)
    y2_ref[...] = dinv * jnp.dot(h, w_ref[...],
                                 preferred_element_type=jnp.float32)


def _tc2(agg_parts, deg_parts, b_row, W2):
    return pl.pallas_call(
        _tc2_body,
        grid=(NRB,),
        in_specs=[
            pl.BlockSpec((NC, RB, H), lambda i: (0, i, 0)),
            pl.BlockSpec((NC, RB, 128), lambda i: (0, i, 0)),
            pl.BlockSpec((1, H), lambda i: (0, 0)),
            pl.BlockSpec((H, H), lambda i: (0, 0)),
        ],
        out_specs=pl.BlockSpec((RB, H), lambda i: (i, 0)),
        out_shape=jax.ShapeDtypeStruct((NP, H), jnp.float32),
    )(agg_parts, deg_parts, b_row, W2)


def _tc3_body(aggp_ref, degp_ref, b_ref, batch_ref, wfc_ref, bfc_ref,
              out_ref, sums, cnts):
    i = pl.program_id(0)

    @pl.when(i == 0)
    def _():
        sums[...] = jnp.zeros_like(sums)
        cnts[...] = jnp.zeros_like(cnts)

    dinv = _dinv_block(degp_ref)
    h = jnp.maximum(dinv * (aggp_ref[0] + aggp_ref[1]) + b_ref[...], 0.0)
    gid = lax.broadcasted_iota(jnp.int32, (RB, 128), 1)
    onehot = (batch_ref[...] == gid).astype(jnp.float32)
    sums[...] += lax.dot_general(onehot, h, (((0,), (0,)), ((), ())),
                                 preferred_element_type=jnp.float32)
    cnts[...] += lax.dot_general(onehot, jnp.ones((RB, 1), jnp.float32),
                                 (((0,), (0,)), ((), ())),
                                 preferred_element_type=jnp.float32)

    @pl.when(i == NRB - 1)
    def _():
        pooled = sums[...] / jnp.maximum(cnts[...], 1.0)
        out_ref[...] = jnp.dot(pooled, wfc_ref[...],
                               preferred_element_type=jnp.float32) + bfc_ref[...]


def _tc3(agg_parts, deg_parts, b_row, batch2, wfc_pad, bfc_row):
    return pl.pallas_call(
        _tc3_body,
        grid=(NRB,),
        in_specs=[
            pl.BlockSpec((NC, RB, H), lambda i: (0, i, 0)),
            pl.BlockSpec((NC, RB, 128), lambda i: (0, i, 0)),
            pl.BlockSpec((1, H), lambda i: (0, 0)),
            pl.BlockSpec((RB, 1), lambda i: (i, 0)),
            pl.BlockSpec((H, 128), lambda i: (0, 0)),
            pl.BlockSpec((1, 128), lambda i: (0, 0)),
        ],
        out_specs=pl.BlockSpec((128, 128), lambda i: (0, 0)),
        out_shape=jax.ShapeDtypeStruct((128, 128), jnp.float32),
        scratch_shapes=[
            pltpu.VMEM((128, 128), jnp.float32),
            pltpu.VMEM((128, 1), jnp.float32),
        ],
    )(agg_parts, deg_parts, b_row, batch2, wfc_pad, bfc_row)


def kernel(x, edge_index, batch, W1, b1, W2, b2, Wfc, bfc):
    x = x.astype(jnp.float32)
    src_flat = edge_index[0].astype(jnp.int32)
    dst_flat = edge_index[1].astype(jnp.int32)
    batch2 = jnp.pad(batch.astype(jnp.int32), (0, NP - N),
                     constant_values=G).reshape(NP, 1)
    x_pad = jnp.pad(x, ((0, NP - N), (0, 0)))
    zeros2 = jnp.zeros((NP, H), jnp.float32)
    ones2 = jnp.ones((CH, 128), jnp.float32)
    wfc_pad = jnp.pad(Wfc.astype(jnp.float32), ((0, 0), (0, 128 - C)))
    bfc_row = jnp.pad(bfc.astype(jnp.float32), (0, 128 - C)).reshape(1, 128)
    b1_row = b1.astype(jnp.float32).reshape(1, H)
    b2_row = b2.astype(jnp.float32).reshape(1, H)

    deg_parts = _sc_deg(dst_flat, zeros2, ones2)
    y1 = _tc1(deg_parts, x_pad, W1.astype(jnp.float32))
    agg1 = _sc_agg(y1, zeros2, src_flat, dst_flat)
    y2 = _tc2(agg1, deg_parts, b1_row, W2.astype(jnp.float32))
    agg2 = _sc_agg(y2, zeros2, src_flat, dst_flat)
    outp = _tc3(agg2, deg_parts, b2_row, batch2, wfc_pad, bfc_row)
    return outp[:G, :C]


# speedup vs baseline: 18.8050x; 1.2213x over previous
import functools

import jax
import jax.numpy as jnp
from jax import lax
from jax.experimental import pallas as pl
from jax.experimental.pallas import tpu as pltpu
from jax.experimental.pallas import tpu_sc as plsc

N = 10000
NP = 10240
E = 320000
D = 128
H = 128
C = 26
G = 64
NC, NS = 2, 16
NW = NC * NS
EPW = E // NW
CH = 80
NCH = EPW // CH
RPS = NP // NS
DW = 8
RB = 256
NRB = NP // RB

_sc_mesh = plsc.VectorSubcoreMesh(core_axis_name="c", subcore_axis_name="s")


def _sc_pass_body(gather, src_hbm, y_hbm, zeros_hbm, ones_hbm, dst_hbm,
                  out0_hbm, out1_hbm, bufs, c, s):
    if gather:
        (srcvA, srcvB, srcvC, dstvA, dstvB, dstvC,
         rowsA, rowsB, rowsC, acc,
         semGA, semGB, semGC, semSA, semSB, semSC) = bufs
    else:
        dstvA, dstvB, rowsA, acc, semSA, semSB = bufs
    for ks in range(NS):
        @pl.when(s == ks)
        def _():
            for k in range(RPS // CH):
                off = ks * RPS + k * CH

                @pl.when(c == 0)
                def _():
                    pltpu.sync_copy(y_hbm.at[pl.ds(off, CH)], rowsA)
                    pltpu.sync_copy(rowsA, acc.at[pl.ds(off, CH)])

                @pl.when(c != 0)
                def _():
                    pltpu.sync_copy(zeros_hbm.at[pl.ds(off, CH)], rowsA)
                    pltpu.sync_copy(rowsA, acc.at[pl.ds(off, CH)])

    if not gather:
        pltpu.sync_copy(ones_hbm, rowsA)
    plsc.subcore_barrier()
    base = (c * NS + s) * EPW

    if gather:
        srcv = (srcvA, srcvB, srcvC)
        dstv = (dstvA, dstvB, dstvC)
        rows = (rowsA, rowsB, rowsC)
        semG = (semGA, semGB, semGC)
        semS = (semSA, semSB, semSC)

        def prefetch(ch, p):
            pltpu.sync_copy(src_hbm.at[pl.ds(base + ch * CH, CH)], srcv[p])
            pltpu.sync_copy(dst_hbm.at[pl.ds(base + ch * CH, CH)], dstv[p])
            pltpu.async_copy(y_hbm.at[srcv[p]], rows[p], semG[p])

        def complete(p):
            pltpu.make_async_copy(y_hbm.at[srcv[p]], rows[p], semG[p]).wait()
            pltpu.async_copy(rows[p], acc.at[dstv[p]], semS[p], add=True)

        def drain_scatter(p):
            pltpu.make_async_copy(rows[p], acc.at[dstv[p]], semS[p]).wait()

        prefetch(0, 0)
        prefetch(1, 1)
        complete(0)
        prefetch(2, 2)

        def body3(j, carry):
            c0 = 3 * j + 1

            def step(q, ch):
                complete(q)
                c2 = ch + 2
                p2 = (q + 2) % 3

                @pl.when(c2 < NCH)
                def _():
                    drain_scatter(p2)
                    prefetch(c2, p2)

            step(1, c0)
            step(2, c0 + 1)
            step(0, c0 + 2)
            return carry

        lax.fori_loop(0, (NCH - 2) // 3, body3, 0)
        complete(1)
        drain_scatter(0)
        drain_scatter(1)
        drain_scatter(2)
    else:
        def loadd(ch, dstv):
            pltpu.sync_copy(dst_hbm.at[pl.ds(base + ch * CH, CH)], dstv)

        loadd(0, dstvA)
        pltpu.async_copy(rowsA, acc.at[dstvA], semSA, add=True)
        loadd(1, dstvB)

        def body2(j, carry):
            pltpu.async_copy(rowsA, acc.at[dstvB], semSB, add=True)
            pltpu.make_async_copy(rowsA, acc.at[dstvA], semSA).wait()

            @pl.when(2 * j + 2 < NCH)
            def _():
                loadd(2 * j + 2, dstvA)
                pltpu.async_copy(rowsA, acc.at[dstvA], semSA, add=True)

            pltpu.make_async_copy(rowsA, acc.at[dstvB], semSB).wait()

            @pl.when(2 * j + 3 < NCH)
            def _():
                loadd(2 * j + 3, dstvB)

            return carry

        lax.fori_loop(0, NCH // 2, body2, 0)
        pltpu.make_async_copy(rowsA, acc.at[dstvA], semSA).wait()

    plsc.subcore_barrier()
    for ks in range(NS):
        @pl.when(s == ks)
        def _():
            for k in range(RPS // CH):
                off = ks * RPS + k * CH
                pltpu.sync_copy(acc.at[pl.ds(off, CH)], rowsA)

                @pl.when(c == 0)
                def _():
                    pltpu.sync_copy(rowsA, out0_hbm.at[pl.ds(off, CH)])

                @pl.when(c != 0)
                def _():
                    pltpu.sync_copy(rowsA, out1_hbm.at[pl.ds(off, CH)])


def _deg_body(dst_hbm, zeros_hbm, ones_hbm, out0_hbm, out1_hbm,
              dstvA, dstvB, rowsA, acc, semSA, semSB):
    c = lax.axis_index("c")
    s = lax.axis_index("s")
    _sc_pass_body(False, None, zeros_hbm, zeros_hbm, ones_hbm, dst_hbm,
                  out0_hbm, out1_hbm, (dstvA, dstvB, rowsA, acc, semSA, semSB),
                  c, s)


def _sc_deg(dst_flat, zeros2, ones2):
    p0, p1 = pl.kernel(
        _deg_body,
        out_type=[jax.ShapeDtypeStruct((NP, 128), jnp.float32),
                  jax.ShapeDtypeStruct((NP, 128), jnp.float32)],
        mesh=_sc_mesh,
        scratch_types=[
            pltpu.VMEM((CH,), jnp.int32),
            pltpu.VMEM((CH,), jnp.int32),
            pltpu.VMEM((CH, 128), jnp.float32),
            pltpu.VMEM_SHARED((NP, 128), jnp.float32),
            pltpu.SemaphoreType.DMA,
            pltpu.SemaphoreType.DMA,
        ],
    )(dst_flat, zeros2, ones2)
    return jnp.stack([p0, p1])


def _agg_body(y_hbm, zeros_hbm, src_hbm, dst_hbm, out0_hbm, out1_hbm,
              srcvA, srcvB, srcvC, dstvA, dstvB, dstvC,
              rowsA, rowsB, rowsC, acc,
              semGA, semGB, semGC, semSA, semSB, semSC):
    c = lax.axis_index("c")
    s = lax.axis_index("s")
    _sc_pass_body(True, src_hbm, y_hbm, zeros_hbm, None, dst_hbm,
                  out0_hbm, out1_hbm,
                  (srcvA, srcvB, srcvC, dstvA, dstvB, dstvC,
                   rowsA, rowsB, rowsC, acc,
                   semGA, semGB, semGC, semSA, semSB, semSC),
                  c, s)


def _sc_agg(y, zeros2, src_flat, dst_flat):
    p0, p1 = pl.kernel(
        _agg_body,
        out_type=[jax.ShapeDtypeStruct((NP, H), jnp.float32),
                  jax.ShapeDtypeStruct((NP, H), jnp.float32)],
        mesh=_sc_mesh,
        scratch_types=(
            [pltpu.VMEM((CH,), jnp.int32)] * 6
            + [pltpu.VMEM((CH, H), jnp.float32)] * 3
            + [pltpu.VMEM_SHARED((NP, H), jnp.float32)]
            + [pltpu.SemaphoreType.DMA] * 6
        ),
    )(y, zeros2, src_flat, dst_flat)
    return jnp.stack([p0, p1])


def _dinv_block(degp_ref):
    deg = degp_ref[0][:, 0:1] + degp_ref[1][:, 0:1]
    return lax.rsqrt(deg + 1.0)


def _tc1_body(degp_ref, x_ref, w_ref, y_ref):
    dinv = _dinv_block(degp_ref)
    y_ref[...] = dinv * jnp.dot(x_ref[...], w_ref[...],
                                preferred_element_type=jnp.float32)


def _tc1(deg_parts, x_pad, W1):
    return pl.pallas_call(
        _tc1_body,
        grid=(NRB,),
        in_specs=[
            pl.BlockSpec((NC, RB, 128), lambda i: (0, i, 0)),
            pl.BlockSpec((RB, D), lambda i: (i, 0)),
            pl.BlockSpec((D, H), lambda i: (0, 0)),
        ],
        out_specs=pl.BlockSpec((RB, H), lambda i: (i, 0)),
        out_shape=jax.ShapeDtypeStruct((NP, H), jnp.float32),
    )(deg_parts, x_pad, W1)


def _tc2_body(aggp_ref, degp_ref, b_ref, w_ref, y2_ref):
    dinv = _dinv_block(degp_ref)
    h = jnp.maximum(dinv * (aggp_ref[0] + aggp_ref[1]) + b_ref[...], 0.0)
    y2_ref[...] = dinv * jnp.dot(h, w_ref[...],
                                 preferred_element_type=jnp.float32)


def _tc2(agg_parts, deg_parts, b_row, W2):
    return pl.pallas_call(
        _tc2_body,
        grid=(NRB,),
        in_specs=[
            pl.BlockSpec((NC, RB, H), lambda i: (0, i, 0)),
            pl.BlockSpec((NC, RB, 128), lambda i: (0, i, 0)),
            pl.BlockSpec((1, H), lambda i: (0, 0)),
            pl.BlockSpec((H, H), lambda i: (0, 0)),
        ],
        out_specs=pl.BlockSpec((RB, H), lambda i: (i, 0)),
        out_shape=jax.ShapeDtypeStruct((NP, H), jnp.float32),
    )(agg_parts, deg_parts, b_row, W2)


def _tc3_body(aggp_ref, degp_ref, b_ref, batch_ref, wfc_ref, bfc_ref,
              out_ref, sums, cnts):
    i = pl.program_id(0)

    @pl.when(i == 0)
    def _():
        sums[...] = jnp.zeros_like(sums)
        cnts[...] = jnp.zeros_like(cnts)

    dinv = _dinv_block(degp_ref)
    h = jnp.maximum(dinv * (aggp_ref[0] + aggp_ref[1]) + b_ref[...], 0.0)
    gid = lax.broadcasted_iota(jnp.int32, (RB, 128), 1)
    onehot = (batch_ref[...] == gid).astype(jnp.float32)
    sums[...] += lax.dot_general(onehot, h, (((0,), (0,)), ((), ())),
                                 preferred_element_type=jnp.float32)
    cnts[...] += lax.dot_general(onehot, jnp.ones((RB, 1), jnp.float32),
                                 (((0,), (0,)), ((), ())),
                                 preferred_element_type=jnp.float32)

    @pl.when(i == NRB - 1)
    def _():
        pooled = sums[...] / jnp.maximum(cnts[...], 1.0)
        out_ref[...] = jnp.dot(pooled, wfc_ref[...],
                               preferred_element_type=jnp.float32) + bfc_ref[...]


def _tc3(agg_parts, deg_parts, b_row, batch2, wfc_pad, bfc_row):
    return pl.pallas_call(
        _tc3_body,
        grid=(NRB,),
        in_specs=[
            pl.BlockSpec((NC, RB, H), lambda i: (0, i, 0)),
            pl.BlockSpec((NC, RB, 128), lambda i: (0, i, 0)),
            pl.BlockSpec((1, H), lambda i: (0, 0)),
            pl.BlockSpec((RB, 1), lambda i: (i, 0)),
            pl.BlockSpec((H, 128), lambda i: (0, 0)),
            pl.BlockSpec((1, 128), lambda i: (0, 0)),
        ],
        out_specs=pl.BlockSpec((128, 128), lambda i: (0, 0)),
        out_shape=jax.ShapeDtypeStruct((128, 128), jnp.float32),
        scratch_shapes=[
            pltpu.VMEM((128, 128), jnp.float32),
            pltpu.VMEM((128, 1), jnp.float32),
        ],
    )(agg_parts, deg_parts, b_row, batch2, wfc_pad, bfc_row)


def kernel(x, edge_index, batch, W1, b1, W2, b2, Wfc, bfc):
    x = x.astype(jnp.float32)
    src_flat = edge_index[0].astype(jnp.int32)
    dst_flat = edge_index[1].astype(jnp.int32)
    batch2 = jnp.pad(batch.astype(jnp.int32), (0, NP - N),
                     constant_values=G).reshape(NP, 1)
    x_pad = jnp.pad(x, ((0, NP - N), (0, 0)))
    zeros2 = jnp.zeros((NP, H), jnp.float32)
    ones2 = jnp.ones((CH, 128), jnp.float32)
    wfc_pad = jnp.pad(Wfc.astype(jnp.float32), ((0, 0), (0, 128 - C)))
    bfc_row = jnp.pad(bfc.astype(jnp.float32), (0, 128 - C)).reshape(1, 128)
    b1_row = b1.astype(jnp.float32).reshape(1, H)
    b2_row = b2.astype(jnp.float32).reshape(1, H)

    deg_parts = _sc_deg(dst_flat, zeros2, ones2)
    y1 = _tc1(deg_parts, x_pad, W1.astype(jnp.float32))
    agg1 = _sc_agg(y1, zeros2, src_flat, dst_flat)
    y2 = _tc2(agg1, deg_parts, b1_row, W2.astype(jnp.float32))
    agg2 = _sc_agg(y2, zeros2, src_flat, dst_flat)
    outp = _tc3(agg2, deg_parts, b2_row, batch2, wfc_pad, bfc_row)
    return outp[:G, :C]


# speedup vs baseline: 18.8203x; 1.0008x over previous
import jax
import jax.numpy as jnp
from jax import lax
from jax.experimental import pallas as pl
from jax.experimental.pallas import tpu as pltpu
from jax.experimental.pallas import tpu_sc as plsc

N = 10000
NP = 10240
E = 320000
D = 128
H = 128
C = 26
G = 64
NC, NS = 2, 16
NW = NC * NS
EPW = E // NW
CH = 80
NCH = EPW // CH
RPS = NP // NS
DW = 8
RB = 256
NRB = NP // RB

_sc_mesh = plsc.VectorSubcoreMesh(core_axis_name="c", subcore_axis_name="s")


def _sc_pass_body(gather, src_hbm, y_hbm, zeros_hbm, ones_hbm, dst_hbm,
                  out0_hbm, out1_hbm, bufs, c, s):
    if gather:
        (srcvA, srcvB, srcvC, dstvA, dstvB, dstvC,
         rowsA, rowsB, rowsC, acc,
         semGA, semGB, semGC, semSA, semSB, semSC) = bufs
    else:
        dstvA, dstvB, rowsA, acc, semSA, semSB = bufs
    for ks in range(NS):
        @pl.when(s == ks)
        def _():
            for k in range(RPS // CH):
                off = ks * RPS + k * CH

                @pl.when(c == 0)
                def _():
                    pltpu.sync_copy(y_hbm.at[pl.ds(off, CH)], rowsA)
                    pltpu.sync_copy(rowsA, acc.at[pl.ds(off, CH)])

                @pl.when(c != 0)
                def _():
                    pltpu.sync_copy(zeros_hbm.at[pl.ds(off, CH)], rowsA)
                    pltpu.sync_copy(rowsA, acc.at[pl.ds(off, CH)])

    if not gather:
        pltpu.sync_copy(ones_hbm, rowsA)
    plsc.subcore_barrier()
    base = (c * NS + s) * EPW

    if gather:
        srcv = (srcvA, srcvB, srcvC)
        dstv = (dstvA, dstvB, dstvC)
        rows = (rowsA, rowsB, rowsC)
        semG = (semGA, semGB, semGC)
        semS = (semSA, semSB, semSC)

        def prefetch(ch, p):
            pltpu.sync_copy(src_hbm.at[pl.ds(base + ch * CH, CH)], srcv[p])
            pltpu.sync_copy(dst_hbm.at[pl.ds(base + ch * CH, CH)], dstv[p])
            pltpu.async_copy(y_hbm.at[srcv[p]], rows[p], semG[p])

        def complete(p):
            pltpu.make_async_copy(y_hbm.at[srcv[p]], rows[p], semG[p]).wait()
            pltpu.async_copy(rows[p], acc.at[dstv[p]], semS[p], add=True)

        def drain_scatter(p):
            pltpu.make_async_copy(rows[p], acc.at[dstv[p]], semS[p]).wait()

        prefetch(0, 0)
        prefetch(1, 1)
        complete(0)
        prefetch(2, 2)

        def body3(j, carry):
            c0 = 3 * j + 1

            def step(q, ch):
                complete(q)
                c2 = ch + 2
                p2 = (q + 2) % 3

                @pl.when(c2 < NCH)
                def _():
                    drain_scatter(p2)
                    prefetch(c2, p2)

            step(1, c0)
            step(2, c0 + 1)
            step(0, c0 + 2)
            return carry

        lax.fori_loop(0, (NCH - 2) // 3, body3, 0)
        complete(1)
        drain_scatter(0)
        drain_scatter(1)
        drain_scatter(2)
    else:
        def loadd(ch, dstv):
            pltpu.sync_copy(dst_hbm.at[pl.ds(base + ch * CH, CH)], dstv)

        loadd(0, dstvA)
        pltpu.async_copy(rowsA, acc.at[dstvA], semSA, add=True)
        loadd(1, dstvB)

        def body2(j, carry):
            pltpu.async_copy(rowsA, acc.at[dstvB], semSB, add=True)
            pltpu.make_async_copy(rowsA, acc.at[dstvA], semSA).wait()

            @pl.when(2 * j + 2 < NCH)
            def _():
                loadd(2 * j + 2, dstvA)
                pltpu.async_copy(rowsA, acc.at[dstvA], semSA, add=True)

            pltpu.make_async_copy(rowsA, acc.at[dstvB], semSB).wait()

            @pl.when(2 * j + 3 < NCH)
            def _():
                loadd(2 * j + 3, dstvB)

            return carry

        lax.fori_loop(0, NCH // 2, body2, 0)
        pltpu.make_async_copy(rowsA, acc.at[dstvA], semSA).wait()

    plsc.subcore_barrier()
    for ks in range(NS):
        @pl.when(s == ks)
        def _():
            for k in range(RPS // CH):
                off = ks * RPS + k * CH
                pltpu.sync_copy(acc.at[pl.ds(off, CH)], rowsA)

                @pl.when(c == 0)
                def _():
                    pltpu.sync_copy(rowsA, out0_hbm.at[pl.ds(off, CH)])

                @pl.when(c != 0)
                def _():
                    pltpu.sync_copy(rowsA, out1_hbm.at[pl.ds(off, CH)])


def _deg_body(dst_hbm, zeros_hbm, ones_hbm, out0_hbm, out1_hbm,
              dstvA, dstvB, rowsA, acc, semSA, semSB):
    c = lax.axis_index("c")
    s = lax.axis_index("s")
    _sc_pass_body(False, None, zeros_hbm, zeros_hbm, ones_hbm, dst_hbm,
                  out0_hbm, out1_hbm, (dstvA, dstvB, rowsA, acc, semSA, semSB),
                  c, s)


def _sc_deg(dst_flat, zeros2, ones2):
    p0, p1 = pl.kernel(
        _deg_body,
        out_type=[jax.ShapeDtypeStruct((NP, 128), jnp.float32),
                  jax.ShapeDtypeStruct((NP, 128), jnp.float32)],
        mesh=_sc_mesh,
        scratch_types=[
            pltpu.VMEM((CH,), jnp.int32),
            pltpu.VMEM((CH,), jnp.int32),
            pltpu.VMEM((CH, 128), jnp.float32),
            pltpu.VMEM_SHARED((NP, 128), jnp.float32),
            pltpu.SemaphoreType.DMA,
            pltpu.SemaphoreType.DMA,
        ],
    )(dst_flat, zeros2, ones2)
    return jnp.stack([p0, p1])


def _agg_body(y_hbm, zeros_hbm, src_hbm, dst_hbm, out0_hbm, out1_hbm,
              srcvA, srcvB, srcvC, dstvA, dstvB, dstvC,
              rowsA, rowsB, rowsC, acc,
              semGA, semGB, semGC, semSA, semSB, semSC):
    c = lax.axis_index("c")
    s = lax.axis_index("s")
    _sc_pass_body(True, src_hbm, y_hbm, zeros_hbm, None, dst_hbm,
                  out0_hbm, out1_hbm,
                  (srcvA, srcvB, srcvC, dstvA, dstvB, dstvC,
                   rowsA, rowsB, rowsC, acc,
                   semGA, semGB, semGC, semSA, semSB, semSC),
                  c, s)


def _sc_agg(y, zeros2, src_flat, dst_flat):
    p0, p1 = pl.kernel(
        _agg_body,
        out_type=[jax.ShapeDtypeStruct((NP, H), jnp.float32),
                  jax.ShapeDtypeStruct((NP, H), jnp.float32)],
        mesh=_sc_mesh,
        scratch_types=(
            [pltpu.VMEM((CH,), jnp.int32)] * 6
            + [pltpu.VMEM((CH, H), jnp.float32)] * 3
            + [pltpu.VMEM_SHARED((NP, H), jnp.float32)]
            + [pltpu.SemaphoreType.DMA] * 6
        ),
    )(y, zeros2, src_flat, dst_flat)
    return jnp.stack([p0, p1])


def _dinv_block(degp_ref):
    deg = degp_ref[0][:, 0:1] + degp_ref[1][:, 0:1]
    return lax.rsqrt(deg + 1.0)


def _tc1_body(degp_ref, x_ref, w_ref, y_ref):
    dinv = _dinv_block(degp_ref)
    y_ref[...] = dinv * jnp.dot(x_ref[...], w_ref[...],
                                preferred_element_type=jnp.float32)


def _tc1(deg_parts, x_pad, W1):
    return pl.pallas_call(
        _tc1_body,
        grid=(NRB,),
        in_specs=[
            pl.BlockSpec((NC, RB, 128), lambda i: (0, i, 0)),
            pl.BlockSpec((RB, D), lambda i: (i, 0)),
            pl.BlockSpec((D, H), lambda i: (0, 0)),
        ],
        out_specs=pl.BlockSpec((RB, H), lambda i: (i, 0)),
        out_shape=jax.ShapeDtypeStruct((NP, H), jnp.float32),
    )(deg_parts, x_pad, W1)


def _tc2_body(aggp_ref, degp_ref, b_ref, w_ref, y2_ref):
    dinv = _dinv_block(degp_ref)
    h = jnp.maximum(dinv * (aggp_ref[0] + aggp_ref[1]) + b_ref[...], 0.0)
    y2_ref[...] = dinv * jnp.dot(h, w_ref[...],
                                 preferred_element_type=jnp.float32)


def _tc2(agg_parts, deg_parts, b_row, W2):
    return pl.pallas_call(
        _tc2_body,
        grid=(NRB,),
        in_specs=[
            pl.BlockSpec((NC, RB, H), lambda i: (0, i, 0)),
            pl.BlockSpec((NC, RB, 128), lambda i: (0, i, 0)),
            pl.BlockSpec((1, H), lambda i: (0, 0)),
            pl.BlockSpec((H, H), lambda i: (0, 0)),
        ],
        out_specs=pl.BlockSpec((RB, H), lambda i: (i, 0)),
        out_shape=jax.ShapeDtypeStruct((NP, H), jnp.float32),
    )(agg_parts, deg_parts, b_row, W2)


def _tc3_body(aggp_ref, degp_ref, b_ref, batch_ref, wfc_ref, bfc_ref,
              out_ref, sums, cnts):
    i = pl.program_id(0)

    @pl.when(i == 0)
    def _():
        sums[...] = jnp.zeros_like(sums)
        cnts[...] = jnp.zeros_like(cnts)

    dinv = _dinv_block(degp_ref)
    h = jnp.maximum(dinv * (aggp_ref[0] + aggp_ref[1]) + b_ref[...], 0.0)
    gid = lax.broadcasted_iota(jnp.int32, (RB, 128), 1)
    onehot = (batch_ref[...] == gid).astype(jnp.float32)
    sums[...] += lax.dot_general(onehot, h, (((0,), (0,)), ((), ())),
                                 preferred_element_type=jnp.float32)
    cnts[...] += lax.dot_general(onehot, jnp.ones((RB, 1), jnp.float32),
                                 (((0,), (0,)), ((), ())),
                                 preferred_element_type=jnp.float32)

    @pl.when(i == NRB - 1)
    def _():
        pooled = sums[...] / jnp.maximum(cnts[...], 1.0)
        out_ref[...] = jnp.dot(pooled, wfc_ref[...],
                               preferred_element_type=jnp.float32) + bfc_ref[...]


def _tc3(agg_parts, deg_parts, b_row, batch2, wfc_pad, bfc_row):
    return pl.pallas_call(
        _tc3_body,
        grid=(NRB,),
        in_specs=[
            pl.BlockSpec((NC, RB, H), lambda i: (0, i, 0)),
            pl.BlockSpec((NC, RB, 128), lambda i: (0, i, 0)),
            pl.BlockSpec((1, H), lambda i: (0, 0)),
            pl.BlockSpec((RB, 1), lambda i: (i, 0)),
            pl.BlockSpec((H, 128), lambda i: (0, 0)),
            pl.BlockSpec((1, 128), lambda i: (0, 0)),
        ],
        out_specs=pl.BlockSpec((128, 128), lambda i: (0, 0)),
        out_shape=jax.ShapeDtypeStruct((128, 128), jnp.float32),
        scratch_shapes=[
            pltpu.VMEM((128, 128), jnp.float32),
            pltpu.VMEM((128, 1), jnp.float32),
        ],
    )(agg_parts, deg_parts, b_row, batch2, wfc_pad, bfc_row)


def kernel(x, edge_index, batch, W1, b1, W2, b2, Wfc, bfc):
    x = x.astype(jnp.float32)
    src_flat = edge_index[0].astype(jnp.int32)
    dst_flat = edge_index[1].astype(jnp.int32)
    batch2 = jnp.pad(batch.astype(jnp.int32), (0, NP - N),
                     constant_values=G).reshape(NP, 1)
    x_pad = jnp.pad(x, ((0, NP - N), (0, 0)))
    zeros2 = jnp.zeros((NP, H), jnp.float32)
    ones2 = jnp.ones((CH, 128), jnp.float32)
    wfc_pad = jnp.pad(Wfc.astype(jnp.float32), ((0, 0), (0, 128 - C)))
    bfc_row = jnp.pad(bfc.astype(jnp.float32), (0, 128 - C)).reshape(1, 128)
    b1_row = b1.astype(jnp.float32).reshape(1, H)
    b2_row = b2.astype(jnp.float32).reshape(1, H)

    deg_parts = _sc_deg(dst_flat, zeros2, ones2)
    y1 = _tc1(deg_parts, x_pad, W1.astype(jnp.float32))
    agg1 = _sc_agg(y1, zeros2, src_flat, dst_flat)
    y2 = _tc2(agg1, deg_parts, b1_row, W2.astype(jnp.float32))
    agg2 = _sc_agg(y2, zeros2, src_flat, dst_flat)
    outp = _tc3(agg2, deg_parts, b2_row, batch2, wfc_pad, bfc_row)
    return outp[:G, :C]
